# Initial kernel scaffold; baseline (speedup 1.0000x reference)
import functools

import jax
import jax.numpy as jnp
from jax import lax
from jax.experimental import pallas as pl
from jax.experimental.pallas import tpu as pltpu
from jax.experimental.pallas import tpu_sc as plsc

N = 10000
E = 320000
D = 128
C = 64
AW = 144
HW = 80
BN = 1000
NC, NS = 2, 16
NW = NC * NS
EW = E // NW
K = 80
NCHUNK = EW // K
RT = 624
REM = N - RT * NS


def _stage_a(x_ref, w1_ref, w2_ref, wr_ref, x1aug_ref, xr_ref):
    x = x_ref[...]
    t = jnp.maximum(jnp.dot(x, w1_ref[...].T, preferred_element_type=jnp.float32), 0.0)
    m = jnp.dot(t, w2_ref[...].T, preferred_element_type=jnp.float32)
    x1 = x * jax.nn.sigmoid(2.0 * m)
    xr_ref[...] = jnp.dot(x1, wr_ref[...].T, preferred_element_type=jnp.float32)
    bn = x1.shape[0]
    x1aug_ref[...] = jnp.concatenate(
        [x1, jnp.ones((bn, 1), jnp.float32), jnp.zeros((bn, AW - D - 1), jnp.float32)],
        axis=1)


def _call_stage_a(x, fb_w1, fb_w2, sage_wr):
    return pl.pallas_call(
        _stage_a,
        grid=(N // BN,),
        in_specs=[
            pl.BlockSpec((BN, D), lambda i: (i, 0)),
            pl.BlockSpec((C, D), lambda i: (0, 0)),
            pl.BlockSpec((D, C), lambda i: (0, 0)),
            pl.BlockSpec((D, D), lambda i: (0, 0)),
        ],
        out_specs=[
            pl.BlockSpec((BN, AW), lambda i: (i, 0)),
            pl.BlockSpec((BN, D), lambda i: (i, 0)),
        ],
        out_shape=[
            jax.ShapeDtypeStruct((N, AW), jnp.float32),
            jax.ShapeDtypeStruct((N, D), jnp.float32),
        ],
    )(x, fb_w1, fb_w2, sage_wr)


def _sage_body(x1aug, src_hbm, dst_hbm, zrows, out,
               src_v, dst_v, rows_v, acc_sh, sem):
    c = lax.axis_index("c")
    s = lax.axis_index("s")
    pltpu.sync_copy(zrows.at[pl.ds(0, RT)], acc_sh.at[pl.ds(s * RT, RT)])

    @pl.when(s == 0)
    def _():
        pltpu.sync_copy(zrows.at[pl.ds(0, REM)], acc_sh.at[pl.ds(RT * NS, REM)])

    plsc.subcore_barrier()

    wid = s * NC + c
    base = wid * EW

    def chunk(i, _):
        off = base + i * K
        pltpu.sync_copy(src_hbm.at[pl.ds(off, K)], src_v)
        pltpu.sync_copy(dst_hbm.at[pl.ds(off, K)], dst_v)
        pltpu.async_copy(x1aug.at[src_v], rows_v, sem).wait()
        pltpu.sync_copy(rows_v, acc_sh.at[dst_v], add=True)
        return 0

    lax.fori_loop(0, NCHUNK, chunk, 0)
    plsc.subcore_barrier()
    pltpu.sync_copy(acc_sh.at[pl.ds(s * RT, RT)], out.at[c, pl.ds(s * RT, RT)])

    @pl.when(s == 0)
    def _():
        pltpu.sync_copy(acc_sh.at[pl.ds(RT * NS, REM)], out.at[c, pl.ds(RT * NS, REM)])


def _call_sage(x1aug, src, dst, zrows):
    return pl.kernel(
        _sage_body,
        out_type=jax.ShapeDtypeStruct((NC, N, AW), jnp.float32),
        mesh=plsc.VectorSubcoreMesh(core_axis_name="c", subcore_axis_name="s"),
        scratch_types=[
            pltpu.VMEM((K,), jnp.int32),
            pltpu.VMEM((K,), jnp.int32),
            pltpu.VMEM((K, AW), jnp.float32),
            pltpu.VMEM_SHARED((N, AW), jnp.float32),
            pltpu.SemaphoreType.DMA,
        ],
    )(x1aug, src, dst, zrows)


def _stage_c(aggdeg_ref, xr_ref, wl_ref, bl_ref, gw_ref, att_ref,
             haug_ref, ad_ref, pmax_ref):
    a = aggdeg_ref[0]
    b = aggdeg_ref[1]
    agg = a[:, :D] + b[:, :D]
    deg = a[:, D] + b[:, D]
    mean = agg / jnp.maximum(deg, 1.0)[:, None]
    x2 = jnp.maximum(
        jnp.dot(mean, wl_ref[...].T, preferred_element_type=jnp.float32)
        + bl_ref[...] + xr_ref[...], 0.0)
    h = jnp.dot(x2, gw_ref[...].T, preferred_element_type=jnp.float32)
    att = att_ref[...]
    a_s = jnp.sum(h * att[0][None, :], axis=1)
    a_d = jnp.sum(h * att[1][None, :], axis=1)
    bn = h.shape[0]
    haug_ref[...] = jnp.concatenate(
        [h, jnp.ones((bn, 1), jnp.float32), a_s[:, None],
         jnp.zeros((bn, HW - C - 2), jnp.float32)], axis=1)
    ad_ref[...] = a_d[:, None]
    pmax_ref[...] = jnp.concatenate(
        [jnp.max(a_s).reshape(1, 1), jnp.max(a_d).reshape(1, 1),
         jnp.zeros((1, 6), jnp.float32)], axis=1)


def _call_stage_c(aggdeg, xr, sage_wl, sage_bl, gat_w, att):
    return pl.pallas_call(
        _stage_c,
        grid=(N // BN,),
        in_specs=[
            pl.BlockSpec((NC, BN, AW), lambda i: (0, i, 0)),
            pl.BlockSpec((BN, D), lambda i: (i, 0)),
            pl.BlockSpec((D, D), lambda i: (0, 0)),
            pl.BlockSpec((1, D), lambda i: (0, 0)),
            pl.BlockSpec((C, D), lambda i: (0, 0)),
            pl.BlockSpec((2, C), lambda i: (0, 0)),
        ],
        out_specs=[
            pl.BlockSpec((BN, HW), lambda i: (i, 0)),
            pl.BlockSpec((BN, 1), lambda i: (i, 0)),
            pl.BlockSpec((1, 8), lambda i: (i, 0)),
        ],
        out_shape=[
            jax.ShapeDtypeStruct((N, HW), jnp.float32),
            jax.ShapeDtypeStruct((N, 1), jnp.float32),
            jax.ShapeDtypeStruct((N // BN, 8), jnp.float32),
        ],
    )(aggdeg, xr, sage_wl, sage_bl, gat_w, att)


def _gat_body(haug, ad_hbm, src_hbm, dst_hbm, shift_hbm, zrows, out,
              src_v, dst_v, hbuf, wbuf, ad_v, shift_v, acc_sh, sem):
    c = lax.axis_index("c")
    s = lax.axis_index("s")
    pltpu.sync_copy(zrows.at[pl.ds(0, RT)], acc_sh.at[pl.ds(s * RT, RT)])

    @pl.when(s == 0)
    def _():
        pltpu.sync_copy(zrows.at[pl.ds(0, REM)], acc_sh.at[pl.ds(RT * NS, REM)])

    pltpu.sync_copy(ad_hbm, ad_v)
    pltpu.sync_copy(shift_hbm, shift_v)
    plsc.subcore_barrier()

    shift = shift_v[...]
    wid = s * NC + c
    base = wid * EW

    def chunk(i, _):
        off = base + i * K
        pltpu.sync_copy(src_hbm.at[pl.ds(off, K)], src_v)
        pltpu.sync_copy(dst_hbm.at[pl.ds(off, K)], dst_v)
        pltpu.async_copy(haug.at[src_v], hbuf, sem).wait()
        for g in range(K // 16):
            rows16 = lax.iota(jnp.int32, 16) + g * 16
            as16 = plsc.load_gather(hbuf, [rows16, jnp.full((16,), C + 1, jnp.int32)])
            didx = dst_v[pl.ds(g * 16, 16)]
            ad16 = plsc.load_gather(ad_v, [didx])
            u = as16 + ad16
            e = jnp.where(u > 0.0, u, 0.2 * u)
            wbuf[pl.ds(g * 16, 16)] = jnp.exp(e - shift)

        def jbody(j, _):
            wj = plsc.load_gather(wbuf, [jnp.full((16,), 0, jnp.int32) + j])
            for cg in range(HW // 16):
                hbuf[j, pl.ds(cg * 16, 16)] = hbuf[j, pl.ds(cg * 16, 16)] * wj
            return 0

        lax.fori_loop(0, K, jbody, 0)
        pltpu.sync_copy(hbuf, acc_sh.at[dst_v], add=True)
        return 0

    lax.fori_loop(0, NCHUNK, chunk, 0)
    plsc.subcore_barrier()
    pltpu.sync_copy(acc_sh.at[pl.ds(s * RT, RT)], out.at[c, pl.ds(s * RT, RT)])

    @pl.when(s == 0)
    def _():
        pltpu.sync_copy(acc_sh.at[pl.ds(RT * NS, REM)], out.at[c, pl.ds(RT * NS, REM)])


def _call_gat(haug, ad, src, dst, shift16, zrows):
    return pl.kernel(
        _gat_body,
        out_type=jax.ShapeDtypeStruct((NC, N, HW), jnp.float32),
        mesh=plsc.VectorSubcoreMesh(core_axis_name="c", subcore_axis_name="s"),
        scratch_types=[
            pltpu.VMEM((K,), jnp.int32),
            pltpu.VMEM((K,), jnp.int32),
            pltpu.VMEM((K, HW), jnp.float32),
            pltpu.VMEM((K,), jnp.float32),
            pltpu.VMEM((N,), jnp.float32),
            pltpu.VMEM((16,), jnp.float32),
            pltpu.VMEM_SHARED((N, HW), jnp.float32),
            pltpu.SemaphoreType.DMA,
        ],
    )(haug, ad, src, dst, shift16, zrows)


def _stage_e(nd_ref, haug_ref, ad_ref, shift_ref, gb_ref, cw_ref, cb_ref, out_ref):
    acc = nd_ref[0] + nd_ref[1]
    numer = acc[:, :C]
    denom = acc[:, C]
    ha = haug_ref[...]
    h = ha[:, :C]
    a_s = ha[:, C + 1]
    a_d = ad_ref[...][:, 0]
    u = a_s + a_d
    e = jnp.where(u > 0.0, u, 0.2 * u)
    ws = jnp.exp(e - shift_ref[0, 0])
    numer = numer + ws[:, None] * h
    denom = denom + ws
    gat = numer / denom[:, None] + gb_ref[...]
    x3 = jnp.maximum(gat, 0.0)
    z = jnp.sum(x3 * cw_ref[...], axis=1, keepdims=True) + cb_ref[0, 0]
    out_ref[...] = jax.nn.sigmoid(z)


def _call_stage_e(nd, haug, ad, shift, gat_b, cheb_w, cheb_b):
    return pl.pallas_call(
        _stage_e,
        grid=(N // BN,),
        in_specs=[
            pl.BlockSpec((NC, BN, HW), lambda i: (0, i, 0)),
            pl.BlockSpec((BN, HW), lambda i: (i, 0)),
            pl.BlockSpec((BN, 1), lambda i: (i, 0)),
            pl.BlockSpec((1, 1), lambda i: (0, 0)),
            pl.BlockSpec((1, C), lambda i: (0, 0)),
            pl.BlockSpec((1, C), lambda i: (0, 0)),
            pl.BlockSpec((1, 1), lambda i: (0, 0)),
        ],
        out_specs=[pl.BlockSpec((BN, 1), lambda i: (i, 0))],
        out_shape=[jax.ShapeDtypeStruct((N, 1), jnp.float32)],
    )(nd, haug, ad, shift, gat_b, cheb_w, cheb_b)


def kernel(x, edge_index, fb_w1, fb_w2, sage_wl, sage_bl, sage_wr,
           gat_w, att_src, att_dst, gat_b, cheb_w, cheb_b):
    src = edge_index[0]
    dst = edge_index[1]

    x1aug, xr = _call_stage_a(x, fb_w1, fb_w2, sage_wr)

    zrows_a = jnp.zeros((RT, AW), jnp.float32)
    aggdeg = _call_sage(x1aug, src, dst, zrows_a)

    haug, ad, pmax = _call_stage_c(
        aggdeg, xr, sage_wl, sage_bl.reshape(1, D), gat_w,
        jnp.stack([att_src, att_dst]))

    ms = jnp.max(pmax, axis=0)
    u = ms[0] + ms[1]
    shift = jnp.where(u > 0.0, u, 0.2 * u)
    shift16 = jnp.full((16,), shift, jnp.float32)

    zrows_d = jnp.zeros((RT, HW), jnp.float32)
    nd = _call_gat(haug, ad[:, 0], src, dst, shift16, zrows_d)

    (out,) = _call_stage_e(nd, haug, ad, shift.reshape(1, 1),
                           gat_b.reshape(1, C), cheb_w, cheb_b.reshape(1, 1))
    return out


# speedup vs baseline: 14.3548x; 14.3548x over previous
import functools

import jax
import jax.numpy as jnp
from jax import lax
from jax.experimental import pallas as pl
from jax.experimental.pallas import tpu as pltpu
from jax.experimental.pallas import tpu_sc as plsc

N = 10000
E = 320000
D = 128
C = 64
AW = 144
HW = 80
BN = 1000
NC, NS = 2, 16
NW = NC * NS
EW = E // NW
K = 80
NCHUNK = EW // K
RT = 624
REM = N - RT * NS


def _stage_a(x_ref, w1_ref, w2_ref, wr_ref, x1aug_ref, xr_ref):
    x = x_ref[...]
    t = jnp.maximum(jnp.dot(x, w1_ref[...].T, preferred_element_type=jnp.float32), 0.0)
    m = jnp.dot(t, w2_ref[...].T, preferred_element_type=jnp.float32)
    x1 = x * jax.nn.sigmoid(2.0 * m)
    xr_ref[...] = jnp.dot(x1, wr_ref[...].T, preferred_element_type=jnp.float32)
    bn = x1.shape[0]
    x1aug_ref[...] = jnp.concatenate(
        [x1, jnp.ones((bn, 1), jnp.float32), jnp.zeros((bn, AW - D - 1), jnp.float32)],
        axis=1)


def _call_stage_a(x, fb_w1, fb_w2, sage_wr):
    return pl.pallas_call(
        _stage_a,
        grid=(N // BN,),
        in_specs=[
            pl.BlockSpec((BN, D), lambda i: (i, 0)),
            pl.BlockSpec((C, D), lambda i: (0, 0)),
            pl.BlockSpec((D, C), lambda i: (0, 0)),
            pl.BlockSpec((D, D), lambda i: (0, 0)),
        ],
        out_specs=[
            pl.BlockSpec((BN, AW), lambda i: (i, 0)),
            pl.BlockSpec((BN, D), lambda i: (i, 0)),
        ],
        out_shape=[
            jax.ShapeDtypeStruct((N, AW), jnp.float32),
            jax.ShapeDtypeStruct((N, D), jnp.float32),
        ],
    )(x, fb_w1, fb_w2, sage_wr)


def _sage_body(x1aug, src_hbm, dst_hbm, zrows, out,
               src_v, dst_v, rows_v, acc_sh, sem):
    c = lax.axis_index("c")
    s = lax.axis_index("s")
    pltpu.sync_copy(zrows.at[pl.ds(0, RT)], acc_sh.at[pl.ds(s * RT, RT)])

    @pl.when(s == 0)
    def _():
        pltpu.sync_copy(zrows.at[pl.ds(0, REM)], acc_sh.at[pl.ds(RT * NS, REM)])

    plsc.subcore_barrier()

    wid = s * NC + c
    base = wid * EW

    def chunk(i, _):
        off = base + i * K
        pltpu.sync_copy(src_hbm.at[pl.ds(off, K)], src_v)
        pltpu.sync_copy(dst_hbm.at[pl.ds(off, K)], dst_v)
        pltpu.async_copy(x1aug.at[src_v], rows_v, sem).wait()
        pltpu.sync_copy(rows_v, acc_sh.at[dst_v], add=True)
        return 0

    lax.fori_loop(0, NCHUNK, chunk, 0)
    plsc.subcore_barrier()
    pltpu.sync_copy(acc_sh.at[pl.ds(s * RT, RT)], out.at[c, pl.ds(s * RT, RT)])

    @pl.when(s == 0)
    def _():
        pltpu.sync_copy(acc_sh.at[pl.ds(RT * NS, REM)], out.at[c, pl.ds(RT * NS, REM)])


def _call_sage(x1aug, src, dst, zrows):
    return pl.kernel(
        _sage_body,
        out_type=jax.ShapeDtypeStruct((NC, N, AW), jnp.float32),
        mesh=plsc.VectorSubcoreMesh(core_axis_name="c", subcore_axis_name="s"),
        scratch_types=[
            pltpu.VMEM((K,), jnp.int32),
            pltpu.VMEM((K,), jnp.int32),
            pltpu.VMEM((K, AW), jnp.float32),
            pltpu.VMEM_SHARED((N, AW), jnp.float32),
            pltpu.SemaphoreType.DMA,
        ],
        compiler_params=pltpu.CompilerParams(use_tc_tiling_on_sc=False, needs_layout_passes=False),
    )(x1aug, src, dst, zrows)


def _stage_c(aggdeg_ref, xr_ref, wl_ref, bl_ref, gw_ref, att_ref,
             haug_ref, ad_ref, pmax_ref):
    a = aggdeg_ref[0]
    b = aggdeg_ref[1]
    agg = a[:, :D] + b[:, :D]
    deg = a[:, D] + b[:, D]
    mean = agg / jnp.maximum(deg, 1.0)[:, None]
    x2 = jnp.maximum(
        jnp.dot(mean, wl_ref[...].T, preferred_element_type=jnp.float32)
        + bl_ref[...] + xr_ref[...], 0.0)
    h = jnp.dot(x2, gw_ref[...].T, preferred_element_type=jnp.float32)
    att = att_ref[...]
    a_s = jnp.sum(h * att[0][None, :], axis=1)
    a_d = jnp.sum(h * att[1][None, :], axis=1)
    bn = h.shape[0]
    haug_ref[...] = jnp.concatenate(
        [h, jnp.ones((bn, 1), jnp.float32),
         jnp.zeros((bn, HW - C - 1), jnp.float32)], axis=1)
    ad_ref[...] = jnp.concatenate([a_s[:, None], a_d[:, None]], axis=1)
    cur = jnp.concatenate(
        [jnp.max(a_s).reshape(1, 1), jnp.max(a_d).reshape(1, 1),
         jnp.full((1, 6), -jnp.inf, jnp.float32)], axis=1)
    prev = jnp.where(pl.program_id(0) == 0,
                     jnp.full((1, 8), -jnp.inf, jnp.float32), pmax_ref[...])
    pmax_ref[...] = jnp.maximum(prev, cur)


def _call_stage_c(aggdeg, xr, sage_wl, sage_bl, gat_w, att):
    return pl.pallas_call(
        _stage_c,
        grid=(N // BN,),
        in_specs=[
            pl.BlockSpec((NC, BN, AW), lambda i: (0, i, 0)),
            pl.BlockSpec((BN, D), lambda i: (i, 0)),
            pl.BlockSpec((D, D), lambda i: (0, 0)),
            pl.BlockSpec((1, D), lambda i: (0, 0)),
            pl.BlockSpec((C, D), lambda i: (0, 0)),
            pl.BlockSpec((2, C), lambda i: (0, 0)),
        ],
        out_specs=[
            pl.BlockSpec((BN, HW), lambda i: (i, 0)),
            pl.BlockSpec((BN, 2), lambda i: (i, 0)),
            pl.BlockSpec((1, 8), lambda i: (0, 0)),
        ],
        out_shape=[
            jax.ShapeDtypeStruct((N, HW), jnp.float32),
            jax.ShapeDtypeStruct((N, 2), jnp.float32),
            jax.ShapeDtypeStruct((1, 8), jnp.float32),
        ],
    )(aggdeg, xr, sage_wl, sage_bl, gat_w, att)


def _gat_body(haug, as_hbm, ad_hbm, src_hbm, dst_hbm, shift_hbm, zrows, out,
              src_v, dst_v, hbuf, wbuf, as_v, ad_v, shift_v, acc_sh, sem):
    c = lax.axis_index("c")
    s = lax.axis_index("s")
    pltpu.sync_copy(zrows.at[pl.ds(0, RT)], acc_sh.at[pl.ds(s * RT, RT)])

    @pl.when(s == 0)
    def _():
        pltpu.sync_copy(zrows.at[pl.ds(0, REM)], acc_sh.at[pl.ds(RT * NS, REM)])

    pltpu.sync_copy(as_hbm, as_v)
    pltpu.sync_copy(ad_hbm, ad_v)
    pltpu.sync_copy(shift_hbm, shift_v)
    plsc.subcore_barrier()

    shift = shift_v[...]
    wid = s * NC + c
    base = wid * EW

    def chunk(i, _):
        off = base + i * K
        pltpu.sync_copy(src_hbm.at[pl.ds(off, K)], src_v)
        pltpu.sync_copy(dst_hbm.at[pl.ds(off, K)], dst_v)
        pltpu.async_copy(haug.at[src_v], hbuf, sem).wait()
        for g in range(K // 16):
            sidx = src_v[pl.ds(g * 16, 16)]
            as16 = plsc.load_gather(as_v, [sidx])
            didx = dst_v[pl.ds(g * 16, 16)]
            ad16 = plsc.load_gather(ad_v, [didx])
            u = as16 + ad16
            e = jnp.where(u > 0.0, u, 0.2 * u)
            wbuf[pl.ds(g * 16, 16)] = jnp.exp(e - shift)

        def jbody(j, _):
            wj = plsc.load_gather(wbuf, [jnp.full((16,), 0, jnp.int32) + j])
            for cg in range(HW // 16):
                hbuf[j, pl.ds(cg * 16, 16)] = hbuf[j, pl.ds(cg * 16, 16)] * wj
            return 0

        lax.fori_loop(0, K, jbody, 0)
        pltpu.sync_copy(hbuf, acc_sh.at[dst_v], add=True)
        return 0

    lax.fori_loop(0, NCHUNK, chunk, 0)
    plsc.subcore_barrier()
    pltpu.sync_copy(acc_sh.at[pl.ds(s * RT, RT)], out.at[c, pl.ds(s * RT, RT)])

    @pl.when(s == 0)
    def _():
        pltpu.sync_copy(acc_sh.at[pl.ds(RT * NS, REM)], out.at[c, pl.ds(RT * NS, REM)])


def _call_gat(haug, a_s, a_d, src, dst, shift16, zrows):
    return pl.kernel(
        _gat_body,
        out_type=jax.ShapeDtypeStruct((NC, N, HW), jnp.float32),
        mesh=plsc.VectorSubcoreMesh(core_axis_name="c", subcore_axis_name="s"),
        scratch_types=[
            pltpu.VMEM((K,), jnp.int32),
            pltpu.VMEM((K,), jnp.int32),
            pltpu.VMEM((K, HW), jnp.float32),
            pltpu.VMEM((K,), jnp.float32),
            pltpu.VMEM((N,), jnp.float32),
            pltpu.VMEM((N,), jnp.float32),
            pltpu.VMEM((16,), jnp.float32),
            pltpu.VMEM_SHARED((N, HW), jnp.float32),
            pltpu.SemaphoreType.DMA,
        ],
        compiler_params=pltpu.CompilerParams(use_tc_tiling_on_sc=False, needs_layout_passes=False),
    )(haug, a_s, a_d, src, dst, shift16, zrows)


def _stage_e(nd_ref, haug_ref, ad_ref, shift_ref, gb_ref, cw_ref, cb_ref, out_ref):
    acc = nd_ref[0] + nd_ref[1]
    numer = acc[:, :C]
    denom = acc[:, C]
    ha = haug_ref[...]
    h = ha[:, :C]
    aux = ad_ref[...]
    u = aux[:, 0] + aux[:, 1]
    e = jnp.where(u > 0.0, u, 0.2 * u)
    ws = jnp.exp(e - shift_ref[0, 0])
    numer = numer + ws[:, None] * h
    denom = denom + ws
    gat = numer / denom[:, None] + gb_ref[...]
    x3 = jnp.maximum(gat, 0.0)
    z = jnp.sum(x3 * cw_ref[...], axis=1, keepdims=True) + cb_ref[0, 0]
    out_ref[...] = jax.nn.sigmoid(z)


def _call_stage_e(nd, haug, ad, shift, gat_b, cheb_w, cheb_b):
    return pl.pallas_call(
        _stage_e,
        grid=(N // BN,),
        in_specs=[
            pl.BlockSpec((NC, BN, HW), lambda i: (0, i, 0)),
            pl.BlockSpec((BN, HW), lambda i: (i, 0)),
            pl.BlockSpec((BN, 2), lambda i: (i, 0)),
            pl.BlockSpec((1, 1), lambda i: (0, 0)),
            pl.BlockSpec((1, C), lambda i: (0, 0)),
            pl.BlockSpec((1, C), lambda i: (0, 0)),
            pl.BlockSpec((1, 1), lambda i: (0, 0)),
        ],
        out_specs=[pl.BlockSpec((BN, 1), lambda i: (i, 0))],
        out_shape=[jax.ShapeDtypeStruct((N, 1), jnp.float32)],
    )(nd, haug, ad, shift, gat_b, cheb_w, cheb_b)


def kernel(x, edge_index, fb_w1, fb_w2, sage_wl, sage_bl, sage_wr,
           gat_w, att_src, att_dst, gat_b, cheb_w, cheb_b):
    src = edge_index[0]
    dst = edge_index[1]

    x1aug, xr = _call_stage_a(x, fb_w1, fb_w2, sage_wr)

    zrows_a = jnp.zeros((RT, AW), jnp.float32)
    aggdeg = _call_sage(x1aug, src, dst, zrows_a)

    haug, aux, pmax = _call_stage_c(
        aggdeg, xr, sage_wl, sage_bl.reshape(1, D), gat_w,
        jnp.stack([att_src, att_dst]))

    ms = pmax[0]
    u = ms[0] + ms[1]
    shift = jnp.where(u > 0.0, u, 0.2 * u)
    shift16 = jnp.full((16,), shift, jnp.float32)

    zrows_d = jnp.zeros((RT, HW), jnp.float32)
    nd = _call_gat(haug, aux[:, 0], aux[:, 1], src, dst, shift16, zrows_d)

    (out,) = _call_stage_e(nd, haug, aux, shift.reshape(1, 1),
                           gat_b.reshape(1, C), cheb_w, cheb_b.reshape(1, 1))
    return out


# speedup vs baseline: 28.1119x; 1.9584x over previous
import functools

import jax
import jax.numpy as jnp
from jax import lax
from jax.experimental import pallas as pl
from jax.experimental.pallas import tpu as pltpu
from jax.experimental.pallas import tpu_sc as plsc

N = 10000
E = 320000
D = 128
C = 64
AW = 144
HW = 80
BN = 1000
NC, NS = 2, 16
NW = NC * NS
EW = E // NW
KB = 40
NCHUNK_B = EW // KB
NBUF = 5
NBLK_B = NCHUNK_B // NBUF
KD = 80
NCHUNK_D = EW // KD
RT = 624
REM = N - RT * NS


def _stage_a(x_ref, w1_ref, w2_ref, wr_ref, x1aug_ref, xr_ref):
    x = x_ref[...]
    t = jnp.maximum(jnp.dot(x, w1_ref[...].T, preferred_element_type=jnp.float32), 0.0)
    m = jnp.dot(t, w2_ref[...].T, preferred_element_type=jnp.float32)
    x1 = x * jax.nn.sigmoid(2.0 * m)
    xr_ref[...] = jnp.dot(x1, wr_ref[...].T, preferred_element_type=jnp.float32)
    bn = x1.shape[0]
    x1aug_ref[...] = jnp.concatenate(
        [x1, jnp.ones((bn, 1), jnp.float32), jnp.zeros((bn, AW - D - 1), jnp.float32)],
        axis=1)


def _call_stage_a(x, fb_w1, fb_w2, sage_wr):
    return pl.pallas_call(
        _stage_a,
        grid=(N // BN,),
        in_specs=[
            pl.BlockSpec((BN, D), lambda i: (i, 0)),
            pl.BlockSpec((C, D), lambda i: (0, 0)),
            pl.BlockSpec((D, C), lambda i: (0, 0)),
            pl.BlockSpec((D, D), lambda i: (0, 0)),
        ],
        out_specs=[
            pl.BlockSpec((BN, AW), lambda i: (i, 0)),
            pl.BlockSpec((BN, D), lambda i: (i, 0)),
        ],
        out_shape=[
            jax.ShapeDtypeStruct((N, AW), jnp.float32),
            jax.ShapeDtypeStruct((N, D), jnp.float32),
        ],
    )(x, fb_w1, fb_w2, sage_wr)


def _sage_body(x1aug, src_hbm, dst_hbm, zrows, out,
               src_d, dst_d, rows, acc_sh, semg, sems, semi):
    c = lax.axis_index("c")
    s = lax.axis_index("s")
    wid = s * NC + c
    pltpu.sync_copy(zrows.at[pl.ds(0, RT)], acc_sh.at[pl.ds(s * RT, RT)])

    @pl.when(s == 0)
    def _():
        pltpu.sync_copy(zrows.at[pl.ds(0, REM)], acc_sh.at[pl.ds(RT * NS, REM)])

    pltpu.async_copy(src_hbm.at[wid, pl.ds(0, NBUF)], src_d.at[0], semi)
    pltpu.async_copy(dst_hbm.at[wid, pl.ds(0, NBUF)], dst_d.at[0], semi)
    plsc.subcore_barrier()

    def block(ii, _):
        p = lax.rem(ii, 2)
        q = 1 - p
        nxt = lax.rem(ii + 1, NBLK_B)
        pltpu.make_async_copy(src_hbm.at[wid, pl.ds(0, NBUF)], src_d.at[p], semi).wait()
        pltpu.make_async_copy(dst_hbm.at[wid, pl.ds(0, NBUF)], dst_d.at[p], semi).wait()
        pltpu.async_copy(src_hbm.at[wid, pl.ds(nxt * NBUF, NBUF)], src_d.at[q], semi)
        pltpu.async_copy(dst_hbm.at[wid, pl.ds(nxt * NBUF, NBUF)], dst_d.at[q], semi)
        gd = [pltpu.async_copy(x1aug.at[src_d.at[p, b]], rows.at[b], semg)
              for b in range(NBUF)]
        sd = []
        for b in range(NBUF):
            gd[b].wait()
            sd.append(pltpu.async_copy(rows.at[b], acc_sh.at[dst_d.at[p, b]],
                                       sems, add=True))
        for b in range(NBUF):
            sd[b].wait()
        return 0

    lax.fori_loop(0, NBLK_B, block, 0)
    pltpu.make_async_copy(src_hbm.at[wid, pl.ds(0, NBUF)], src_d.at[lax.rem(NBLK_B, 2)], semi).wait()
    pltpu.make_async_copy(dst_hbm.at[wid, pl.ds(0, NBUF)], dst_d.at[lax.rem(NBLK_B, 2)], semi).wait()
    plsc.subcore_barrier()
    pltpu.sync_copy(acc_sh.at[pl.ds(s * RT, RT)], out.at[c, pl.ds(s * RT, RT)])

    @pl.when(s == 0)
    def _():
        pltpu.sync_copy(acc_sh.at[pl.ds(RT * NS, REM)], out.at[c, pl.ds(RT * NS, REM)])


def _call_sage(x1aug, src, dst, zrows):
    return pl.kernel(
        _sage_body,
        out_type=jax.ShapeDtypeStruct((NC, N, AW), jnp.float32),
        mesh=plsc.VectorSubcoreMesh(core_axis_name="c", subcore_axis_name="s"),
        scratch_types=[
            pltpu.VMEM((2, NBUF, KB), jnp.int32),
            pltpu.VMEM((2, NBUF, KB), jnp.int32),
            pltpu.VMEM((NBUF, KB, AW), jnp.float32),
            pltpu.VMEM_SHARED((N, AW), jnp.float32),
            pltpu.SemaphoreType.DMA,
            pltpu.SemaphoreType.DMA,
            pltpu.SemaphoreType.DMA,
        ],
        compiler_params=pltpu.CompilerParams(use_tc_tiling_on_sc=False, needs_layout_passes=False),
    )(x1aug, src, dst, zrows)


def _stage_c(aggdeg_ref, xr_ref, wl_ref, bl_ref, gw_ref, att_ref,
             haug_ref, ad_ref, pmax_ref):
    a = aggdeg_ref[0]
    b = aggdeg_ref[1]
    agg = a[:, :D] + b[:, :D]
    deg = a[:, D] + b[:, D]
    mean = agg / jnp.maximum(deg, 1.0)[:, None]
    x2 = jnp.maximum(
        jnp.dot(mean, wl_ref[...].T, preferred_element_type=jnp.float32)
        + bl_ref[...] + xr_ref[...], 0.0)
    h = jnp.dot(x2, gw_ref[...].T, preferred_element_type=jnp.float32)
    att = att_ref[...]
    a_s = jnp.sum(h * att[0][None, :], axis=1)
    a_d = jnp.sum(h * att[1][None, :], axis=1)
    bn = h.shape[0]
    haug_ref[...] = jnp.concatenate(
        [h, jnp.ones((bn, 1), jnp.float32),
         jnp.zeros((bn, HW - C - 1), jnp.float32)], axis=1)
    ad_ref[...] = jnp.concatenate([a_s[:, None], a_d[:, None]], axis=1)
    cur = jnp.concatenate(
        [jnp.max(a_s).reshape(1, 1), jnp.max(a_d).reshape(1, 1),
         jnp.full((1, 6), -jnp.inf, jnp.float32)], axis=1)
    prev = jnp.where(pl.program_id(0) == 0,
                     jnp.full((1, 8), -jnp.inf, jnp.float32), pmax_ref[...])
    pmax_ref[...] = jnp.maximum(prev, cur)


def _call_stage_c(aggdeg, xr, sage_wl, sage_bl, gat_w, att):
    return pl.pallas_call(
        _stage_c,
        grid=(N // BN,),
        in_specs=[
            pl.BlockSpec((NC, BN, AW), lambda i: (0, i, 0)),
            pl.BlockSpec((BN, D), lambda i: (i, 0)),
            pl.BlockSpec((D, D), lambda i: (0, 0)),
            pl.BlockSpec((1, D), lambda i: (0, 0)),
            pl.BlockSpec((C, D), lambda i: (0, 0)),
            pl.BlockSpec((2, C), lambda i: (0, 0)),
        ],
        out_specs=[
            pl.BlockSpec((BN, HW), lambda i: (i, 0)),
            pl.BlockSpec((BN, 2), lambda i: (i, 0)),
            pl.BlockSpec((1, 8), lambda i: (0, 0)),
        ],
        out_shape=[
            jax.ShapeDtypeStruct((N, HW), jnp.float32),
            jax.ShapeDtypeStruct((N, 2), jnp.float32),
            jax.ShapeDtypeStruct((1, 8), jnp.float32),
        ],
    )(aggdeg, xr, sage_wl, sage_bl, gat_w, att)


def _gat_body(haug, as_hbm, ad_hbm, src_hbm, dst_hbm, shift_hbm, zrows, out,
              srcs, dsts, hbuf, wbuf, as_v, ad_v, shift_v, acc_sh, semg, sems):
    c = lax.axis_index("c")
    s = lax.axis_index("s")
    wid = s * NC + c
    pltpu.sync_copy(src_hbm.at[wid], srcs)
    pltpu.sync_copy(dst_hbm.at[wid], dsts)
    pltpu.sync_copy(zrows.at[pl.ds(0, RT)], acc_sh.at[pl.ds(s * RT, RT)])

    @pl.when(s == 0)
    def _():
        pltpu.sync_copy(zrows.at[pl.ds(0, REM)], acc_sh.at[pl.ds(RT * NS, REM)])

    pltpu.sync_copy(as_hbm, as_v)
    pltpu.sync_copy(ad_hbm, ad_v)
    pltpu.sync_copy(shift_hbm, shift_v)
    plsc.subcore_barrier()

    shift = shift_v[...]

    def block(ii, _):
        i0 = ii * NBUF
        gd = [pltpu.async_copy(haug.at[srcs.at[i0 + b]], hbuf.at[b], semg)
              for b in range(NBUF)]
        sd = []
        for b in range(NBUF):
            gd[b].wait()
            for g in range(KD // 16):
                sidx = srcs[i0 + b, pl.ds(g * 16, 16)]
                as16 = plsc.load_gather(as_v, [sidx])
                didx = dsts[i0 + b, pl.ds(g * 16, 16)]
                ad16 = plsc.load_gather(ad_v, [didx])
                u = as16 + ad16
                e = jnp.where(u > 0.0, u, 0.2 * u)
                wbuf[pl.ds(g * 16, 16)] = jnp.exp(e - shift)

            def jbody(j, _):
                wj = plsc.load_gather(wbuf, [jnp.full((16,), 0, jnp.int32) + j])
                for cg in range(HW // 16):
                    hbuf[b, j, pl.ds(cg * 16, 16)] = (
                        hbuf[b, j, pl.ds(cg * 16, 16)] * wj)
                return 0

            lax.fori_loop(0, KD, jbody, 0)
            sd.append(pltpu.async_copy(hbuf.at[b], acc_sh.at[dsts.at[i0 + b]],
                                       sems, add=True))
        for b in range(NBUF):
            sd[b].wait()
        return 0

    lax.fori_loop(0, NCHUNK_D // NBUF, block, 0)
    plsc.subcore_barrier()
    pltpu.sync_copy(acc_sh.at[pl.ds(s * RT, RT)], out.at[c, pl.ds(s * RT, RT)])

    @pl.when(s == 0)
    def _():
        pltpu.sync_copy(acc_sh.at[pl.ds(RT * NS, REM)], out.at[c, pl.ds(RT * NS, REM)])


def _call_gat(haug, a_s, a_d, src, dst, shift16, zrows):
    return pl.kernel(
        _gat_body,
        out_type=jax.ShapeDtypeStruct((NC, N, HW), jnp.float32),
        mesh=plsc.VectorSubcoreMesh(core_axis_name="c", subcore_axis_name="s"),
        scratch_types=[
            pltpu.VMEM((NCHUNK_D, KD), jnp.int32),
            pltpu.VMEM((NCHUNK_D, KD), jnp.int32),
            pltpu.VMEM((NBUF, KD, HW), jnp.float32),
            pltpu.VMEM((KD,), jnp.float32),
            pltpu.VMEM((N,), jnp.float32),
            pltpu.VMEM((N,), jnp.float32),
            pltpu.VMEM((16,), jnp.float32),
            pltpu.VMEM_SHARED((N, HW), jnp.float32),
            pltpu.SemaphoreType.DMA,
            pltpu.SemaphoreType.DMA,
        ],
        compiler_params=pltpu.CompilerParams(use_tc_tiling_on_sc=False, needs_layout_passes=False),
    )(haug, a_s, a_d, src, dst, shift16, zrows)


def _stage_e(nd_ref, haug_ref, ad_ref, shift_ref, gb_ref, cw_ref, cb_ref, out_ref):
    acc = nd_ref[0] + nd_ref[1]
    numer = acc[:, :C]
    denom = acc[:, C]
    ha = haug_ref[...]
    h = ha[:, :C]
    aux = ad_ref[...]
    u = aux[:, 0] + aux[:, 1]
    e = jnp.where(u > 0.0, u, 0.2 * u)
    ws = jnp.exp(e - shift_ref[0, 0])
    numer = numer + ws[:, None] * h
    denom = denom + ws
    gat = numer / denom[:, None] + gb_ref[...]
    x3 = jnp.maximum(gat, 0.0)
    z = jnp.sum(x3 * cw_ref[...], axis=1, keepdims=True) + cb_ref[0, 0]
    out_ref[...] = jax.nn.sigmoid(z)


def _call_stage_e(nd, haug, ad, shift, gat_b, cheb_w, cheb_b):
    return pl.pallas_call(
        _stage_e,
        grid=(N // BN,),
        in_specs=[
            pl.BlockSpec((NC, BN, HW), lambda i: (0, i, 0)),
            pl.BlockSpec((BN, HW), lambda i: (i, 0)),
            pl.BlockSpec((BN, 2), lambda i: (i, 0)),
            pl.BlockSpec((1, 1), lambda i: (0, 0)),
            pl.BlockSpec((1, C), lambda i: (0, 0)),
            pl.BlockSpec((1, C), lambda i: (0, 0)),
            pl.BlockSpec((1, 1), lambda i: (0, 0)),
        ],
        out_specs=[pl.BlockSpec((BN, 1), lambda i: (i, 0))],
        out_shape=[jax.ShapeDtypeStruct((N, 1), jnp.float32)],
    )(nd, haug, ad, shift, gat_b, cheb_w, cheb_b)


def kernel(x, edge_index, fb_w1, fb_w2, sage_wl, sage_bl, sage_wr,
           gat_w, att_src, att_dst, gat_b, cheb_w, cheb_b):
    src_b = edge_index[0].reshape(NW, NCHUNK_B, KB)
    dst_b = edge_index[1].reshape(NW, NCHUNK_B, KB)
    src_d = edge_index[0].reshape(NW, NCHUNK_D, KD)
    dst_d = edge_index[1].reshape(NW, NCHUNK_D, KD)

    x1aug, xr = _call_stage_a(x, fb_w1, fb_w2, sage_wr)

    zrows_a = jnp.zeros((RT, AW), jnp.float32)
    aggdeg = _call_sage(x1aug, src_b, dst_b, zrows_a)

    haug, aux, pmax = _call_stage_c(
        aggdeg, xr, sage_wl, sage_bl.reshape(1, D), gat_w,
        jnp.stack([att_src, att_dst]))

    ms = pmax[0]
    u = ms[0] + ms[1]
    shift = jnp.where(u > 0.0, u, 0.2 * u)
    shift16 = jnp.full((16,), shift, jnp.float32)

    zrows_d = jnp.zeros((RT, HW), jnp.float32)
    nd = _call_gat(haug, aux[:, 0], aux[:, 1], src_d, dst_d, shift16, zrows_d)

    (out,) = _call_stage_e(nd, haug, aux, shift.reshape(1, 1),
                           gat_b.reshape(1, C), cheb_w, cheb_b.reshape(1, 1))
    return out


# speedup vs baseline: 30.9302x; 1.1003x over previous
import functools

import jax
import jax.numpy as jnp
from jax import lax
from jax.experimental import pallas as pl
from jax.experimental.pallas import tpu as pltpu
from jax.experimental.pallas import tpu_sc as plsc

N = 10000
E = 320000
D = 128
C = 64
AW = 144
HW = 80
BN = 1000
NC, NS = 2, 16
NW = NC * NS
EW = E // NW
KB = 40
NCHUNK_B = EW // KB
NBUF = 5
NBLK_B = NCHUNK_B // NBUF
KD = 80
NCHUNK_D = EW // KD
RT = 624
REM = N - RT * NS


def _stage_a(x_ref, w1_ref, w2_ref, wr_ref, x1aug_ref, xr_ref):
    x = x_ref[...]
    t = jnp.maximum(jnp.dot(x, w1_ref[...].T, preferred_element_type=jnp.float32), 0.0)
    m = jnp.dot(t, w2_ref[...].T, preferred_element_type=jnp.float32)
    x1 = x * jax.nn.sigmoid(2.0 * m)
    xr_ref[...] = jnp.dot(x1, wr_ref[...].T, preferred_element_type=jnp.float32)
    bn = x1.shape[0]
    x1aug_ref[...] = jnp.concatenate(
        [x1, jnp.ones((bn, 1), jnp.float32), jnp.zeros((bn, AW - D - 1), jnp.float32)],
        axis=1)


def _call_stage_a(x, fb_w1, fb_w2, sage_wr):
    return pl.pallas_call(
        _stage_a,
        grid=(N // BN,),
        in_specs=[
            pl.BlockSpec((BN, D), lambda i: (i, 0)),
            pl.BlockSpec((C, D), lambda i: (0, 0)),
            pl.BlockSpec((D, C), lambda i: (0, 0)),
            pl.BlockSpec((D, D), lambda i: (0, 0)),
        ],
        out_specs=[
            pl.BlockSpec((BN, AW), lambda i: (i, 0)),
            pl.BlockSpec((BN, D), lambda i: (i, 0)),
        ],
        out_shape=[
            jax.ShapeDtypeStruct((N, AW), jnp.float32),
            jax.ShapeDtypeStruct((N, D), jnp.float32),
        ],
    )(x, fb_w1, fb_w2, sage_wr)


def _sage_body(x1aug, src_hbm, dst_hbm, zrows, out,
               src_d, dst_d, rows, acc_sh, semg, sems, semi):
    c = lax.axis_index("c")
    s = lax.axis_index("s")
    wid = s * NC + c
    pltpu.sync_copy(zrows.at[pl.ds(0, RT)], acc_sh.at[pl.ds(s * RT, RT)])

    @pl.when(s == 0)
    def _():
        pltpu.sync_copy(zrows.at[pl.ds(0, REM)], acc_sh.at[pl.ds(RT * NS, REM)])

    pltpu.async_copy(src_hbm.at[wid, pl.ds(0, NBUF)], src_d.at[0], semi)
    pltpu.async_copy(dst_hbm.at[wid, pl.ds(0, NBUF)], dst_d.at[0], semi)
    plsc.subcore_barrier()

    def block(ii, _):
        p = lax.rem(ii, 2)
        q = 1 - p
        nxt = lax.rem(ii + 1, NBLK_B)
        pltpu.make_async_copy(src_hbm.at[wid, pl.ds(0, NBUF)], src_d.at[p], semi).wait()
        pltpu.make_async_copy(dst_hbm.at[wid, pl.ds(0, NBUF)], dst_d.at[p], semi).wait()
        pltpu.async_copy(src_hbm.at[wid, pl.ds(nxt * NBUF, NBUF)], src_d.at[q], semi)
        pltpu.async_copy(dst_hbm.at[wid, pl.ds(nxt * NBUF, NBUF)], dst_d.at[q], semi)
        gd = [pltpu.async_copy(x1aug.at[src_d.at[p, b]], rows.at[b], semg)
              for b in range(NBUF)]
        sd = []
        for b in range(NBUF):
            gd[b].wait()
            sd.append(pltpu.async_copy(rows.at[b], acc_sh.at[dst_d.at[p, b]],
                                       sems, add=True))
        for b in range(NBUF):
            sd[b].wait()
        return 0

    lax.fori_loop(0, NBLK_B, block, 0)
    pltpu.make_async_copy(src_hbm.at[wid, pl.ds(0, NBUF)], src_d.at[lax.rem(NBLK_B, 2)], semi).wait()
    pltpu.make_async_copy(dst_hbm.at[wid, pl.ds(0, NBUF)], dst_d.at[lax.rem(NBLK_B, 2)], semi).wait()
    plsc.subcore_barrier()
    pltpu.sync_copy(acc_sh.at[pl.ds(s * RT, RT)], out.at[c, pl.ds(s * RT, RT)])

    @pl.when(s == 0)
    def _():
        pltpu.sync_copy(acc_sh.at[pl.ds(RT * NS, REM)], out.at[c, pl.ds(RT * NS, REM)])


def _call_sage(x1aug, src, dst, zrows):
    return pl.kernel(
        _sage_body,
        out_type=jax.ShapeDtypeStruct((NC, N, AW), jnp.float32),
        mesh=plsc.VectorSubcoreMesh(core_axis_name="c", subcore_axis_name="s"),
        scratch_types=[
            pltpu.VMEM((2, NBUF, KB), jnp.int32),
            pltpu.VMEM((2, NBUF, KB), jnp.int32),
            pltpu.VMEM((NBUF, KB, AW), jnp.float32),
            pltpu.VMEM_SHARED((N, AW), jnp.float32),
            pltpu.SemaphoreType.DMA,
            pltpu.SemaphoreType.DMA,
            pltpu.SemaphoreType.DMA,
        ],
        compiler_params=pltpu.CompilerParams(use_tc_tiling_on_sc=False, needs_layout_passes=False),
    )(x1aug, src, dst, zrows)


def _stage_c(aggdeg_ref, xr_ref, wl_ref, bl_ref, gw_ref, att_ref,
             haug_ref, ad_ref, pmax_ref):
    a = aggdeg_ref[0]
    b = aggdeg_ref[1]
    agg = a[:, :D] + b[:, :D]
    deg = a[:, D] + b[:, D]
    mean = agg / jnp.maximum(deg, 1.0)[:, None]
    x2 = jnp.maximum(
        jnp.dot(mean, wl_ref[...].T, preferred_element_type=jnp.float32)
        + bl_ref[...] + xr_ref[...], 0.0)
    h = jnp.dot(x2, gw_ref[...].T, preferred_element_type=jnp.float32)
    att = att_ref[...]
    a_s = jnp.sum(h * att[0][None, :], axis=1)
    a_d = jnp.sum(h * att[1][None, :], axis=1)
    bn = h.shape[0]
    haug_ref[...] = jnp.concatenate(
        [h, jnp.ones((bn, 1), jnp.float32),
         jnp.zeros((bn, HW - C - 1), jnp.float32)], axis=1)
    ad_ref[...] = jnp.concatenate([a_s[:, None], a_d[:, None]], axis=1)
    cur = jnp.concatenate(
        [jnp.max(a_s).reshape(1, 1), jnp.max(a_d).reshape(1, 1),
         jnp.full((1, 6), -jnp.inf, jnp.float32)], axis=1)
    prev = jnp.where(pl.program_id(0) == 0,
                     jnp.full((1, 8), -jnp.inf, jnp.float32), pmax_ref[...])
    pmax_ref[...] = jnp.maximum(prev, cur)


def _call_stage_c(aggdeg, xr, sage_wl, sage_bl, gat_w, att):
    return pl.pallas_call(
        _stage_c,
        grid=(N // BN,),
        in_specs=[
            pl.BlockSpec((NC, BN, AW), lambda i: (0, i, 0)),
            pl.BlockSpec((BN, D), lambda i: (i, 0)),
            pl.BlockSpec((D, D), lambda i: (0, 0)),
            pl.BlockSpec((1, D), lambda i: (0, 0)),
            pl.BlockSpec((C, D), lambda i: (0, 0)),
            pl.BlockSpec((2, C), lambda i: (0, 0)),
        ],
        out_specs=[
            pl.BlockSpec((BN, HW), lambda i: (i, 0)),
            pl.BlockSpec((BN, 2), lambda i: (i, 0)),
            pl.BlockSpec((1, 8), lambda i: (0, 0)),
        ],
        out_shape=[
            jax.ShapeDtypeStruct((N, HW), jnp.float32),
            jax.ShapeDtypeStruct((N, 2), jnp.float32),
            jax.ShapeDtypeStruct((1, 8), jnp.float32),
        ],
    )(aggdeg, xr, sage_wl, sage_bl, gat_w, att)


def _gat_body(haug, as_hbm, ad_hbm, src_hbm, dst_hbm, shift_hbm, zrows, out,
              srcs, dsts, hbuf, wbuf, as_v, ad_v, shift_v, acc_sh, semg, sems):
    c = lax.axis_index("c")
    s = lax.axis_index("s")
    wid = s * NC + c
    pltpu.sync_copy(src_hbm.at[wid], srcs)
    pltpu.sync_copy(dst_hbm.at[wid], dsts)
    pltpu.sync_copy(zrows.at[pl.ds(0, RT)], acc_sh.at[pl.ds(s * RT, RT)])

    @pl.when(s == 0)
    def _():
        pltpu.sync_copy(zrows.at[pl.ds(0, REM)], acc_sh.at[pl.ds(RT * NS, REM)])

    pltpu.sync_copy(as_hbm, as_v)
    pltpu.sync_copy(ad_hbm, ad_v)
    pltpu.sync_copy(shift_hbm, shift_v)
    plsc.subcore_barrier()

    shift = shift_v[...]

    def block(ii, _):
        i0 = ii * NBUF
        gd = [pltpu.async_copy(haug.at[srcs.at[i0 + b]], hbuf.at[b], semg)
              for b in range(NBUF)]
        sd = []
        for b in range(NBUF):
            gd[b].wait()
            for g in range(KD // 16):
                sidx = srcs[i0 + b, pl.ds(g * 16, 16)]
                as16 = plsc.load_gather(as_v, [sidx])
                didx = dsts[i0 + b, pl.ds(g * 16, 16)]
                ad16 = plsc.load_gather(ad_v, [didx])
                u = as16 + ad16
                e = jnp.where(u > 0.0, u, 0.2 * u)
                wbuf[pl.ds(g * 16, 16)] = jnp.exp(e - shift)

            @plsc.parallel_loop(0, KD, unroll=4)
            def jbody(j):
                wj = plsc.load_gather(wbuf, [jnp.full((16,), 0, jnp.int32) + j])
                for cg in range(HW // 16):
                    hbuf[b, j, pl.ds(cg * 16, 16)] = (
                        hbuf[b, j, pl.ds(cg * 16, 16)] * wj)
            sd.append(pltpu.async_copy(hbuf.at[b], acc_sh.at[dsts.at[i0 + b]],
                                       sems, add=True))
        for b in range(NBUF):
            sd[b].wait()
        return 0

    lax.fori_loop(0, NCHUNK_D // NBUF, block, 0)
    plsc.subcore_barrier()
    pltpu.sync_copy(acc_sh.at[pl.ds(s * RT, RT)], out.at[c, pl.ds(s * RT, RT)])

    @pl.when(s == 0)
    def _():
        pltpu.sync_copy(acc_sh.at[pl.ds(RT * NS, REM)], out.at[c, pl.ds(RT * NS, REM)])


def _call_gat(haug, a_s, a_d, src, dst, shift16, zrows):
    return pl.kernel(
        _gat_body,
        out_type=jax.ShapeDtypeStruct((NC, N, HW), jnp.float32),
        mesh=plsc.VectorSubcoreMesh(core_axis_name="c", subcore_axis_name="s"),
        scratch_types=[
            pltpu.VMEM((NCHUNK_D, KD), jnp.int32),
            pltpu.VMEM((NCHUNK_D, KD), jnp.int32),
            pltpu.VMEM((NBUF, KD, HW), jnp.float32),
            pltpu.VMEM((KD,), jnp.float32),
            pltpu.VMEM((N,), jnp.float32),
            pltpu.VMEM((N,), jnp.float32),
            pltpu.VMEM((16,), jnp.float32),
            pltpu.VMEM_SHARED((N, HW), jnp.float32),
            pltpu.SemaphoreType.DMA,
            pltpu.SemaphoreType.DMA,
        ],
        compiler_params=pltpu.CompilerParams(use_tc_tiling_on_sc=False, needs_layout_passes=False),
    )(haug, a_s, a_d, src, dst, shift16, zrows)


def _stage_e(nd_ref, haug_ref, ad_ref, shift_ref, gb_ref, cw_ref, cb_ref, out_ref):
    acc = nd_ref[0] + nd_ref[1]
    numer = acc[:, :C]
    denom = acc[:, C]
    ha = haug_ref[...]
    h = ha[:, :C]
    aux = ad_ref[...]
    u = aux[:, 0] + aux[:, 1]
    e = jnp.where(u > 0.0, u, 0.2 * u)
    ws = jnp.exp(e - shift_ref[0, 0])
    numer = numer + ws[:, None] * h
    denom = denom + ws
    gat = numer / denom[:, None] + gb_ref[...]
    x3 = jnp.maximum(gat, 0.0)
    z = jnp.sum(x3 * cw_ref[...], axis=1, keepdims=True) + cb_ref[0, 0]
    out_ref[...] = jax.nn.sigmoid(z)


def _call_stage_e(nd, haug, ad, shift, gat_b, cheb_w, cheb_b):
    return pl.pallas_call(
        _stage_e,
        grid=(N // BN,),
        in_specs=[
            pl.BlockSpec((NC, BN, HW), lambda i: (0, i, 0)),
            pl.BlockSpec((BN, HW), lambda i: (i, 0)),
            pl.BlockSpec((BN, 2), lambda i: (i, 0)),
            pl.BlockSpec((1, 1), lambda i: (0, 0)),
            pl.BlockSpec((1, C), lambda i: (0, 0)),
            pl.BlockSpec((1, C), lambda i: (0, 0)),
            pl.BlockSpec((1, 1), lambda i: (0, 0)),
        ],
        out_specs=[pl.BlockSpec((BN, 1), lambda i: (i, 0))],
        out_shape=[jax.ShapeDtypeStruct((N, 1), jnp.float32)],
    )(nd, haug, ad, shift, gat_b, cheb_w, cheb_b)


def kernel(x, edge_index, fb_w1, fb_w2, sage_wl, sage_bl, sage_wr,
           gat_w, att_src, att_dst, gat_b, cheb_w, cheb_b):
    src_b = edge_index[0].reshape(NW, NCHUNK_B, KB)
    dst_b = edge_index[1].reshape(NW, NCHUNK_B, KB)
    src_d = edge_index[0].reshape(NW, NCHUNK_D, KD)
    dst_d = edge_index[1].reshape(NW, NCHUNK_D, KD)

    x1aug, xr = _call_stage_a(x, fb_w1, fb_w2, sage_wr)

    zrows_a = jnp.zeros((RT, AW), jnp.float32)
    aggdeg = _call_sage(x1aug, src_b, dst_b, zrows_a)

    haug, aux, pmax = _call_stage_c(
        aggdeg, xr, sage_wl, sage_bl.reshape(1, D), gat_w,
        jnp.stack([att_src, att_dst]))

    ms = pmax[0]
    u = ms[0] + ms[1]
    shift = jnp.where(u > 0.0, u, 0.2 * u)
    shift16 = jnp.full((16,), shift, jnp.float32)

    zrows_d = jnp.zeros((RT, HW), jnp.float32)
    nd = _call_gat(haug, aux[:, 0], aux[:, 1], src_d, dst_d, shift16, zrows_d)

    (out,) = _call_stage_e(nd, haug, aux, shift.reshape(1, 1),
                           gat_b.reshape(1, C), cheb_w, cheb_b.reshape(1, 1))
    return out
